# baseline (device time: 104253 ns/iter reference)
import jax
import jax.numpy as jnp
from jax import lax
from jax.experimental import pallas as pl
from jax.experimental.pallas import tpu as pltpu

N_DEV = 32
B = 2
SQ = 512
D = 1024
H_LOC = 8
DH = 128
ROWS = B * SQ
CHUNK = ROWS // N_DEV
SCALE = 0.08838834764831843


def kernel(x, Wq, Wk, Wv, Wo):
    x2d = x.reshape(ROWS, D)

    def body(x_ref, wq_ref, wk_ref, wv_ref, wo_ref, out_ref,
             q_scr, k_scr, v_scr, partial_ref, p16_ref, gather_ref,
             red_ref, out16_ref, send_sems, recv_sems1, recv_sems2):
        my = lax.axis_index("i")

        bsem = pltpu.get_barrier_semaphore()
        for j in range(1, N_DEV):
            pl.semaphore_signal(bsem, inc=1,
                                device_id=(lax.rem(my + j, N_DEV),),
                                device_id_type=pl.DeviceIdType.MESH)
        pl.semaphore_wait(bsem, N_DEV - 1)

        def p1_desc(j):
            t = lax.rem(my + j, N_DEV)
            return pltpu.make_async_remote_copy(
                src_ref=p16_ref.at[pl.ds(t * CHUNK, CHUNK), :],
                dst_ref=gather_ref.at[j],
                send_sem=send_sems.at[j],
                recv_sem=recv_sems1.at[j],
                device_id=(t,),
                device_id_type=pl.DeviceIdType.MESH,
            )

        pos = lax.broadcasted_iota(jnp.int32, (SQ, DH), 0).astype(jnp.float32)
        dcol = lax.broadcasted_iota(jnp.int32, (SQ, DH), 1)
        d_even = (dcol - lax.rem(dcol, 2)).astype(jnp.float32)
        inv = jnp.exp(d_even * (-jnp.log(10000.0) / DH))
        ang = pos * inv
        cos_s = jnp.cos(ang)
        sin_s = jnp.sin(ang)
        even = lax.rem(dcol, 2) == 0

        def rot(t):
            r_m = jnp.concatenate([t[:, 1:], t[:, :1]], axis=1)
            r_p = jnp.concatenate([t[:, -1:], t[:, :-1]], axis=1)
            return jnp.where(even, -r_m, r_p)

        xv = x_ref[...]
        for b in range(B):
            r0 = b * SQ
            xb = xv[r0:r0 + SQ]
            q_scr[pl.ds(r0, SQ), :] = jnp.dot(
                xb, wq_ref[...], preferred_element_type=jnp.float32)
            k_scr[pl.ds(r0, SQ), :] = jnp.dot(
                xb, wk_ref[...], preferred_element_type=jnp.float32)
            v_scr[pl.ds(r0, SQ), :] = jnp.dot(
                xb, wv_ref[...], preferred_element_type=jnp.float32)
            for h in range(H_LOC):
                c0 = h * DH
                q = q_scr[pl.ds(r0, SQ), pl.ds(c0, DH)]
                k = k_scr[pl.ds(r0, SQ), pl.ds(c0, DH)]
                v = v_scr[pl.ds(r0, SQ), pl.ds(c0, DH)]
                qr = q * cos_s + rot(q) * sin_s
                kr = k * cos_s + rot(k) * sin_s
                s = lax.dot_general(
                    qr, kr, (((1,), (1,)), ((), ())),
                    preferred_element_type=jnp.float32) * SCALE
                m = jnp.max(s, axis=1, keepdims=True)
                w = jnp.exp(s - m)
                w = w / jnp.sum(w, axis=1, keepdims=True)
                q_scr[pl.ds(r0, SQ), pl.ds(c0, DH)] = jnp.dot(
                    w, v, preferred_element_type=jnp.float32)

            partial_ref[pl.ds(r0, SQ), :] = jnp.dot(
                q_scr[pl.ds(r0, SQ), :], wo_ref[...],
                preferred_element_type=jnp.float32)
            p16_ref[pl.ds(r0, SQ), :] = (
                partial_ref[pl.ds(r0, SQ), :].astype(jnp.bfloat16))

            for j in range(1, N_DEV):
                t = lax.rem(my + j, N_DEV)

                @pl.when(lax.div(t, N_DEV // B) == b)
                def _(j=j):
                    p1_desc(j).start()

        p1 = [p1_desc(j) for j in range(1, N_DEV)]
        for d in p1:
            d.wait_recv()

        total = partial_ref[pl.ds(my * CHUNK, CHUNK), :] + jnp.sum(
            gather_ref[pl.ds(1, N_DEV - 1)].astype(jnp.float32), axis=0)
        red_ref[...] = total.astype(jnp.bfloat16)
        out_ref[pl.ds(my * CHUNK, CHUNK), :] = total

        for d in p1:
            d.wait_send()

        def p2_send(j):
            t = lax.rem(my + j, N_DEV)
            return pltpu.make_async_remote_copy(
                src_ref=red_ref,
                dst_ref=out16_ref.at[pl.ds(my * CHUNK, CHUNK), :],
                send_sem=send_sems.at[j],
                recv_sem=recv_sems2.at[j],
                device_id=(t,),
                device_id_type=pl.DeviceIdType.MESH,
            )

        def p2_recv(j):
            s = lax.rem(my - j + N_DEV, N_DEV)
            return pltpu.make_async_remote_copy(
                src_ref=red_ref,
                dst_ref=out16_ref.at[pl.ds(s * CHUNK, CHUNK), :],
                send_sem=send_sems.at[j],
                recv_sem=recv_sems2.at[j],
                device_id=(s,),
                device_id_type=pl.DeviceIdType.MESH,
            )

        p2s = [p2_send(j) for j in range(1, N_DEV)]
        for d in p2s:
            d.start()
        for j in range(1, N_DEV):
            p2_recv(j).wait_recv()
            s = lax.rem(my - j + N_DEV, N_DEV)
            out_ref[pl.ds(s * CHUNK, CHUNK), :] = (
                out16_ref[pl.ds(s * CHUNK, CHUNK), :].astype(jnp.float32))
        for d in p2s:
            d.wait_send()

    out = pl.pallas_call(
        body,
        out_shape=jax.ShapeDtypeStruct((ROWS, D), jnp.float32),
        in_specs=[pl.BlockSpec(memory_space=pltpu.VMEM)] * 5,
        out_specs=pl.BlockSpec(memory_space=pltpu.VMEM),
        scratch_shapes=[
            pltpu.VMEM((ROWS, D), jnp.float32),
            pltpu.VMEM((ROWS, D), jnp.float32),
            pltpu.VMEM((ROWS, D), jnp.float32),
            pltpu.VMEM((ROWS, D), jnp.float32),
            pltpu.VMEM((ROWS, D), jnp.bfloat16),
            pltpu.VMEM((N_DEV, CHUNK, D), jnp.bfloat16),
            pltpu.VMEM((CHUNK, D), jnp.bfloat16),
            pltpu.VMEM((ROWS, D), jnp.bfloat16),
            pltpu.SemaphoreType.DMA((N_DEV,)),
            pltpu.SemaphoreType.DMA((N_DEV,)),
            pltpu.SemaphoreType.DMA((N_DEV,)),
        ],
        compiler_params=pltpu.CompilerParams(collective_id=0),
    )(x2d, Wq, Wk, Wv, Wo)
    return out.reshape(B, SQ, D)


# device time: 98341 ns/iter; 1.0601x vs baseline; 1.0601x over previous
import jax
import jax.numpy as jnp
from jax import lax
from jax.experimental import pallas as pl
from jax.experimental.pallas import tpu as pltpu

N_DEV = 32
B = 2
SQ = 512
D = 1024
H_LOC = 8
DH = 128
ROWS = B * SQ
CHUNK = ROWS // N_DEV
SCALE = 0.08838834764831843


def kernel(x, Wq, Wk, Wv, Wo):
    x2d = x.reshape(ROWS, D)

    def body(x_ref, wq_ref, wk_ref, wv_ref, wo_ref, out_ref,
             q_scr, k_scr, v_scr, partial_ref, p16_ref, gather_ref,
             red_ref, out16_ref, send_sems, recv_sems1, recv_sems2):
        my = lax.axis_index("i")

        xv = x_ref[...]
        q_scr[...] = jnp.dot(xv, wq_ref[...], preferred_element_type=jnp.float32)
        k_scr[...] = jnp.dot(xv, wk_ref[...], preferred_element_type=jnp.float32)
        v_scr[...] = jnp.dot(xv, wv_ref[...], preferred_element_type=jnp.float32)

        pos = lax.broadcasted_iota(jnp.int32, (SQ, DH), 0).astype(jnp.float32)
        dcol = lax.broadcasted_iota(jnp.int32, (SQ, DH), 1)
        d_even = (dcol - lax.rem(dcol, 2)).astype(jnp.float32)
        inv = jnp.exp(d_even * (-jnp.log(10000.0) / DH))
        ang = pos * inv
        cos_s = jnp.cos(ang)
        sin_s = jnp.sin(ang)
        even = lax.rem(dcol, 2) == 0

        def rot(t):
            r_m = jnp.concatenate([t[:, 1:], t[:, :1]], axis=1)
            r_p = jnp.concatenate([t[:, -1:], t[:, :-1]], axis=1)
            return jnp.where(even, -r_m, r_p)

        for b in range(B):
            r0 = b * SQ
            for h in range(H_LOC):
                c0 = h * DH
                q = q_scr[pl.ds(r0, SQ), pl.ds(c0, DH)]
                k = k_scr[pl.ds(r0, SQ), pl.ds(c0, DH)]
                v = v_scr[pl.ds(r0, SQ), pl.ds(c0, DH)]
                qr = q * cos_s + rot(q) * sin_s
                kr = k * cos_s + rot(k) * sin_s
                s = lax.dot_general(
                    qr, kr, (((1,), (1,)), ((), ())),
                    preferred_element_type=jnp.float32) * SCALE
                m = jnp.max(s, axis=1, keepdims=True)
                w = jnp.exp(s - m)
                w = w / jnp.sum(w, axis=1, keepdims=True)
                q_scr[pl.ds(r0, SQ), pl.ds(c0, DH)] = jnp.dot(
                    w, v, preferred_element_type=jnp.float32)

        partial_ref[...] = jnp.dot(q_scr[...], wo_ref[...],
                                   preferred_element_type=jnp.float32)
        p16_ref[...] = partial_ref[...].astype(jnp.bfloat16)

        bsem = pltpu.get_barrier_semaphore()
        for j in range(1, N_DEV):
            pl.semaphore_signal(bsem, inc=1,
                                device_id=(lax.rem(my + j, N_DEV),),
                                device_id_type=pl.DeviceIdType.MESH)
        pl.semaphore_wait(bsem, N_DEV - 1)

        def p1_desc(j):
            t = lax.rem(my + j, N_DEV)
            return pltpu.make_async_remote_copy(
                src_ref=p16_ref.at[pl.ds(t * CHUNK, CHUNK), :],
                dst_ref=gather_ref.at[j],
                send_sem=send_sems.at[j],
                recv_sem=recv_sems1.at[j],
                device_id=(t,),
                device_id_type=pl.DeviceIdType.MESH,
            )

        p1 = [p1_desc(j) for j in range(1, N_DEV)]
        for d in p1:
            d.start()

        total = partial_ref[pl.ds(my * CHUNK, CHUNK), :]
        for j, d in enumerate(p1, start=1):
            d.wait_recv()
            total = total + gather_ref[j].astype(jnp.float32)
        red_ref[...] = total.astype(jnp.bfloat16)

        for d in p1:
            d.wait_send()

        def p2_send(j):
            t = lax.rem(my + j, N_DEV)
            return pltpu.make_async_remote_copy(
                src_ref=red_ref,
                dst_ref=out16_ref.at[pl.ds(my * CHUNK, CHUNK), :],
                send_sem=send_sems.at[j],
                recv_sem=recv_sems2.at[j],
                device_id=(t,),
                device_id_type=pl.DeviceIdType.MESH,
            )

        def p2_recv(j):
            s = lax.rem(my - j + N_DEV, N_DEV)
            return pltpu.make_async_remote_copy(
                src_ref=red_ref,
                dst_ref=out16_ref.at[pl.ds(s * CHUNK, CHUNK), :],
                send_sem=send_sems.at[j],
                recv_sem=recv_sems2.at[j],
                device_id=(s,),
                device_id_type=pl.DeviceIdType.MESH,
            )

        p2s = [p2_send(j) for j in range(1, N_DEV)]
        for d in p2s:
            d.start()
        out_ref[pl.ds(my * CHUNK, CHUNK), :] = total
        for j in range(1, N_DEV):
            p2_recv(j).wait_recv()
            s = lax.rem(my - j + N_DEV, N_DEV)
            out_ref[pl.ds(s * CHUNK, CHUNK), :] = (
                out16_ref[pl.ds(s * CHUNK, CHUNK), :].astype(jnp.float32))
        for d in p2s:
            d.wait_send()

    out = pl.pallas_call(
        body,
        out_shape=jax.ShapeDtypeStruct((ROWS, D), jnp.float32),
        in_specs=[pl.BlockSpec(memory_space=pltpu.VMEM)] * 5,
        out_specs=pl.BlockSpec(memory_space=pltpu.VMEM),
        scratch_shapes=[
            pltpu.VMEM((ROWS, D), jnp.float32),
            pltpu.VMEM((ROWS, D), jnp.float32),
            pltpu.VMEM((ROWS, D), jnp.float32),
            pltpu.VMEM((ROWS, D), jnp.float32),
            pltpu.VMEM((ROWS, D), jnp.bfloat16),
            pltpu.VMEM((N_DEV, CHUNK, D), jnp.bfloat16),
            pltpu.VMEM((CHUNK, D), jnp.bfloat16),
            pltpu.VMEM((ROWS, D), jnp.bfloat16),
            pltpu.SemaphoreType.DMA((N_DEV,)),
            pltpu.SemaphoreType.DMA((N_DEV,)),
            pltpu.SemaphoreType.DMA((N_DEV,)),
        ],
        compiler_params=pltpu.CompilerParams(collective_id=0),
    )(x2d, Wq, Wk, Wv, Wo)
    return out.reshape(B, SQ, D)
